# Initial kernel scaffold; baseline (speedup 1.0000x reference)
import jax
import jax.numpy as jnp
from jax.experimental import pallas as pl


def kernel(target_emb, precursor_flat, cu_seqlens):
    raise NotImplementedError("write your pallas kernel here")


# speedup vs baseline: 3.1824x; 3.1824x over previous
import functools

import jax
import jax.numpy as jnp
from jax import lax
from jax.experimental import pallas as pl
from jax.experimental.pallas import tpu as pltpu
from jax.experimental.pallas import tpu_sc as plsc

B = 16384
D = 256
T = 65536
MAXP = 9
L = 16
NC = 2
NS = 16
NW = NC * NS
BPW = B // NW
NB = 32
NCHUNK = BPW // NB


def _body(tgt_hbm, prec_hbm, cu_hbm, out_hbm,
          cu_v, idx_v, w_v, rows_v, tgt_v, out_v, sem, tsem):
    wid = lax.axis_index("s") * NC + lax.axis_index("c")
    wbase = wid * BPW
    pltpu.sync_copy(cu_hbm.at[pl.ds(pl.multiple_of(wbase, 8), BPW + 32)], cu_v)
    iota = lax.iota(jnp.int32, L)

    def chunk_body(ch, carry):
        cbase = pl.multiple_of(wbase + ch * NB, 8)
        tcopy = pltpu.async_copy(tgt_hbm.at[pl.ds(cbase, NB)], tgt_v, tsem)
        for g in range(NB // L):
            loc = iota + (ch * NB + g * L)
            s = plsc.load_gather(cu_v, [loc])
            cnt = plsc.load_gather(cu_v, [loc + 1]) - s
            for j in range(MAXP):
                idx_v[j, pl.ds(g * L, L)] = jnp.minimum(s + j, T - 1)
                w_v[pl.ds(j * NB + g * L, L)] = jnp.where(
                    cnt > j, jnp.float32(1.0), jnp.float32(0.0))
        cps = [pltpu.async_copy(prec_hbm.at[idx_v.at[j]], rows_v.at[j], sem)
               for j in range(MAXP)]
        tcopy.wait()
        for cp in cps:
            cp.wait()

        def b_body(b, bcarry):
            ws = [plsc.load_gather(w_v, [jnp.full((L,), j * NB, jnp.int32) + b])
                  for j in range(MAXP)]
            for dc in range(D // L):
                acc = tgt_v[b, pl.ds(dc * L, L)]
                for j in range(MAXP):
                    acc = acc + ws[j] * rows_v[j, b, pl.ds(dc * L, L)]
                out_v[b, pl.ds(dc * L, L)] = acc * jnp.float32(0.1)
            return bcarry

        lax.fori_loop(0, NB, b_body, 0)
        pltpu.sync_copy(out_v, out_hbm.at[pl.ds(cbase, NB)])
        return carry

    lax.fori_loop(0, NCHUNK, chunk_body, 0)


@functools.partial(
    pl.kernel,
    out_type=jax.ShapeDtypeStruct((B, D), jnp.float32),
    mesh=plsc.VectorSubcoreMesh(core_axis_name="c", subcore_axis_name="s"),
    scratch_types=[
        pltpu.VMEM((BPW + 32,), jnp.int32),
        pltpu.VMEM((MAXP, NB), jnp.int32),
        pltpu.VMEM((MAXP * NB,), jnp.float32),
        pltpu.VMEM((MAXP, NB, D), jnp.float32),
        pltpu.VMEM((NB, D), jnp.float32),
        pltpu.VMEM((NB, D), jnp.float32),
        pltpu.SemaphoreType.DMA,
        pltpu.SemaphoreType.DMA,
    ],
    compiler_params=pltpu.CompilerParams(needs_layout_passes=False),
)
def _sc_kernel(tgt_hbm, prec_hbm, cu_hbm, out_hbm, *rest):
    _body(tgt_hbm, prec_hbm, cu_hbm, out_hbm, *rest)


def kernel(target_emb, precursor_flat, cu_seqlens):
    cu_pad = jnp.pad(cu_seqlens, (0, 63), mode="edge")
    return _sc_kernel(target_emb, precursor_flat, cu_pad)


# speedup vs baseline: 5.1780x; 1.6271x over previous
import functools

import jax
import jax.numpy as jnp
from jax import lax
from jax.experimental import pallas as pl
from jax.experimental.pallas import tpu as pltpu
from jax.experimental.pallas import tpu_sc as plsc

B = 16384
D = 256
T = 65536
MAXP = 9
L = 16
NC = 2
NS = 16
NW = NC * NS
BPW = B // NW
NB = 16
NCHUNK = BPW // NB


def _body(tgt_hbm, prec_hbm, cu_hbm, out_hbm,
          cu_v, idx_v0, idx_v1, cnt_v, rows_v0, rows_v1, tgt_v0, tgt_v1,
          out_v, gsem0, gsem1, tsem0, tsem1):
    wid = lax.axis_index("s") * NC + lax.axis_index("c")
    wbase = wid * BPW
    idx_vs = (idx_v0, idx_v1)
    rows_vs = (rows_v0, rows_v1)
    tgt_vs = (tgt_v0, tgt_v1)
    gsems = (gsem0, gsem1)
    tsems = (tsem0, tsem1)
    pltpu.sync_copy(cu_hbm.at[pl.ds(pl.multiple_of(wbase, 8), BPW + 32)], cu_v)
    iota = lax.iota(jnp.int32, L)

    def compute_meta(ch, p):
        s = plsc.load_gather(cu_v, [iota + ch * NB])
        cnt = plsc.load_gather(cu_v, [iota + (ch * NB + 1)]) - s
        cnt_v[pl.ds(p * NB, L)] = jnp.minimum(cnt, MAXP)
        for j in range(MAXP):
            idx_vs[p][j, pl.ds(0, L)] = jnp.minimum(s + j, T - 1)

    def copies(ch, p):
        cbase = pl.multiple_of(wbase + ch * NB, 8)
        cps = [pltpu.make_async_copy(
            tgt_hbm.at[pl.ds(cbase, NB)], tgt_vs[p], tsems[p])]
        cps += [pltpu.make_async_copy(
            prec_hbm.at[idx_vs[p].at[j]], rows_vs[p].at[j], gsems[p])
            for j in range(MAXP)]
        return cps

    def fire(ch, p):
        for cp in copies(ch, p):
            cp.start()

    def drain(ch, p):
        for cp in copies(ch, p):
            cp.wait()

    def reduce_out(ch, p):
        def b_body(b, carry):
            cb = plsc.load_gather(
                cnt_v, [jnp.full((L,), p * NB, jnp.int32) + b])[0]
            accs = [tgt_vs[p][b, pl.ds(dc * L, L)] for dc in range(D // L)]

            def j_body(j, accs):
                return [accs[dc] + rows_vs[p][j, b, pl.ds(dc * L, L)]
                        for dc in range(D // L)]

            accs = lax.fori_loop(0, cb, j_body, accs)
            for dc in range(D // L):
                out_v[b, pl.ds(dc * L, L)] = accs[dc] * jnp.float32(0.1)
            return carry

        lax.fori_loop(0, NB, b_body, 0)
        cbase = pl.multiple_of(wbase + ch * NB, 8)
        pltpu.sync_copy(out_v, out_hbm.at[pl.ds(cbase, NB)])

    compute_meta(0, 0)
    fire(0, 0)

    def loop_body(i2, carry):
        ch0 = i2 * 2
        compute_meta(ch0 + 1, 1)
        fire(ch0 + 1, 1)
        drain(ch0, 0)
        reduce_out(ch0, 0)

        @pl.when(i2 < NCHUNK // 2 - 1)
        def _():
            compute_meta(ch0 + 2, 0)
            fire(ch0 + 2, 0)

        drain(ch0 + 1, 1)
        reduce_out(ch0 + 1, 1)
        return carry

    lax.fori_loop(0, NCHUNK // 2, loop_body, 0)


@functools.partial(
    pl.kernel,
    out_type=jax.ShapeDtypeStruct((B, D), jnp.float32),
    mesh=plsc.VectorSubcoreMesh(core_axis_name="c", subcore_axis_name="s"),
    scratch_types=[
        pltpu.VMEM((BPW + 32,), jnp.int32),
        pltpu.VMEM((MAXP, NB), jnp.int32),
        pltpu.VMEM((MAXP, NB), jnp.int32),
        pltpu.VMEM((2 * NB,), jnp.int32),
        pltpu.VMEM((MAXP, NB, D), jnp.float32),
        pltpu.VMEM((MAXP, NB, D), jnp.float32),
        pltpu.VMEM((NB, D), jnp.float32),
        pltpu.VMEM((NB, D), jnp.float32),
        pltpu.VMEM((NB, D), jnp.float32),
        pltpu.SemaphoreType.DMA,
        pltpu.SemaphoreType.DMA,
        pltpu.SemaphoreType.DMA,
        pltpu.SemaphoreType.DMA,
    ],
    compiler_params=pltpu.CompilerParams(needs_layout_passes=False),
)
def _sc_kernel(tgt_hbm, prec_hbm, cu_hbm, out_hbm, *rest):
    _body(tgt_hbm, prec_hbm, cu_hbm, out_hbm, *rest)


def kernel(target_emb, precursor_flat, cu_seqlens):
    cu_pad = jnp.pad(cu_seqlens, (0, 63), mode="edge")
    return _sc_kernel(target_emb, precursor_flat, cu_pad)
